# baseline (speedup 1.0000x reference)
import functools

import jax
import jax.numpy as jnp
from jax import lax
from jax.experimental import pallas as pl
from jax.experimental.pallas import tpu as pltpu
from jax.experimental.pallas import tpu_sc as plsc

_NB = 32
_BATCH = 16384
_NC = 2
_NS = 16
_NW = _NC * _NS
_B_PER_W = _BATCH // _NW
_G = 128
_L = 4
_S = 2 * _L
_H1G = _B_PER_W // _G
_N2 = _B_PER_W * _NB
_H2G = _N2 // _G


def _fused_body(ent_hbm, adj_e_hbm, adj_r_hbm,
                e1_out, e2_out, r1_out, r2_out,
                idx_v, ent1_v, rel1_v, idx2_v, ebuf, rbuf, *sems):
  gsems, wsems, h1sem_e, h1sem_r, h1wsem = (
      sems[:_S], sems[_S:2 * _S], sems[2 * _S], sems[2 * _S + 1],
      sems[2 * _S + 2])
  wid = lax.axis_index("s") * _NC + lax.axis_index("c")
  base = wid * _B_PER_W
  base2 = wid * _N2
  pltpu.sync_copy(ent_hbm.at[pl.ds(base, _B_PER_W)], idx_v)

  h1e, h1r = [], []
  for i in range(_H1G):
    off = idx_v.at[pl.ds(i * _G, _G)]
    dst = pl.ds(i * _G, _G)
    h1e.append(pltpu.async_copy(adj_e_hbm.at[off], ent1_v.at[dst], h1sem_e))
    h1r.append(pltpu.async_copy(adj_r_hbm.at[off], rel1_v.at[dst], h1sem_r))
  for c in h1e:
    c.wait()
  w_e1 = pltpu.async_copy(ent1_v, e1_out.at[pl.ds(base, _B_PER_W)], h1wsem)

  def flat_body(r, carry):
    row = ent1_v.at[r]
    idx2_v[pl.ds(r * _NB, 16)] = row[pl.ds(0, 16)]
    idx2_v[pl.ds(r * _NB + 16, 16)] = row[pl.ds(16, 16)]
    return carry

  lax.fori_loop(0, _B_PER_W, flat_body, 0)

  for c in h1r:
    c.wait()
  w_r1 = pltpu.async_copy(rel1_v, r1_out.at[pl.ds(base, _B_PER_W)], h1wsem)

  def fire_gathers(g, slot):
    off = idx2_v.at[pl.ds(g * _G, _G)]
    pltpu.async_copy(adj_e_hbm.at[off], ebuf.at[slot], gsems[slot])
    pltpu.async_copy(adj_r_hbm.at[off], rbuf.at[slot], gsems[slot])

  def wait_gathers(g, slot):
    out0 = base2 + g * _G
    pltpu.make_async_copy(
        e2_out.at[pl.ds(out0, _G)], ebuf.at[slot], gsems[slot]).wait()
    pltpu.make_async_copy(
        r2_out.at[pl.ds(out0, _G)], rbuf.at[slot], gsems[slot]).wait()

  def fire_writes(g, slot):
    out0 = base2 + g * _G
    pltpu.async_copy(ebuf.at[slot], e2_out.at[pl.ds(out0, _G)], wsems[slot])
    pltpu.async_copy(rbuf.at[slot], r2_out.at[pl.ds(out0, _G)], wsems[slot])

  def wait_writes(g, slot):
    out0 = base2 + g * _G
    pltpu.make_async_copy(
        ebuf.at[slot], e2_out.at[pl.ds(out0, _G)], wsems[slot]).wait()
    pltpu.make_async_copy(
        rbuf.at[slot], r2_out.at[pl.ds(out0, _G)], wsems[slot]).wait()

  for b in range(_L):
    fire_gathers(b, b)
  for b in range(_S):
    g = b
    slot_n = (b + _L) % _S
    if g + _L >= _S:
      wait_writes(g - _L, slot_n)
    fire_gathers(g + _L, slot_n)
    wait_gathers(g, b)
    fire_writes(g, b)

  def outer(t, carry):
    for b in range(_S):
      g = t * _S + b
      slot_n = (b + _L) % _S
      wait_writes(g - _L, slot_n)
      fire_gathers(g + _L, slot_n)
      wait_gathers(g, b)
      fire_writes(g, b)
    return carry

  lax.fori_loop(1, _H2G // _S - 1, outer, 0)

  t_last = _H2G // _S - 1
  for b in range(_S):
    g = t_last * _S + b
    slot_n = (b + _L) % _S
    if g + _L < _H2G:
      wait_writes(g - _L, slot_n)
      fire_gathers(g + _L, slot_n)
    wait_gathers(g, b)
    fire_writes(g, b)
  for b in range(_S):
    wait_writes(_H2G - _S + b, b)
  w_e1.wait()
  w_r1.wait()


def kernel(entity, adj_entity, adj_relation):
  mesh = plsc.VectorSubcoreMesh(
      core_axis_name="c", subcore_axis_name="s",
      num_cores=_NC, num_subcores=_NS)
  out_type = (
      jax.ShapeDtypeStruct((_BATCH, _NB), jnp.int32),
      jax.ShapeDtypeStruct((_BATCH * _NB, _NB), jnp.int32),
      jax.ShapeDtypeStruct((_BATCH, _NB), jnp.int32),
      jax.ShapeDtypeStruct((_BATCH * _NB, _NB), jnp.int32),
  )
  scratch = [
      pltpu.VMEM((_B_PER_W,), jnp.int32),
      pltpu.VMEM((_B_PER_W, _NB), jnp.int32),
      pltpu.VMEM((_B_PER_W, _NB), jnp.int32),
      pltpu.VMEM((_N2,), jnp.int32),
      pltpu.VMEM((_S, _G, _NB), jnp.int32),
      pltpu.VMEM((_S, _G, _NB), jnp.int32),
  ] + [pltpu.SemaphoreType.DMA] * (2 * _S + 3)
  ent1, ent2, rel1, rel2 = pl.kernel(
      _fused_body, out_type=out_type, mesh=mesh, scratch_types=scratch,
      compiler_params=pltpu.CompilerParams(use_tc_tiling_on_sc=False),
  )(entity.reshape(-1), adj_entity, adj_relation)
  return (entity,
          ent1,
          ent2.reshape(_BATCH, _NB * _NB),
          rel1,
          rel2.reshape(_BATCH, _NB * _NB))


# speedup vs baseline: 1.0585x; 1.0585x over previous
import functools

import jax
import jax.numpy as jnp
from jax import lax
from jax.experimental import pallas as pl
from jax.experimental.pallas import tpu as pltpu
from jax.experimental.pallas import tpu_sc as plsc

_NB = 32
_BATCH = 16384
_NC = 2
_NS = 16
_NW = _NC * _NS
_G = 128
_L = 4
_S = 2 * _L


def _mesh():
  return plsc.VectorSubcoreMesh(
      core_axis_name="c", subcore_axis_name="s",
      num_cores=_NC, num_subcores=_NS)


def _ring_pipeline(ng, fire_gathers, wait_gathers, fire_writes, wait_writes):
  if ng <= _S:
    for g in range(ng):
      fire_gathers(g, g)
    for g in range(ng):
      wait_gathers(g, g)
      fire_writes(g, g)
    for g in range(ng):
      wait_writes(g, g)
    return

  for b in range(_L):
    fire_gathers(b, b)
  for b in range(_S):
    g = b
    slot_n = (b + _L) % _S
    if g + _L >= _S:
      wait_writes(g - _L, slot_n)
    fire_gathers(g + _L, slot_n)
    wait_gathers(g, b)
    fire_writes(g, b)

  def outer(t, carry):
    for b in range(_S):
      g = t * _S + b
      slot_n = (b + _L) % _S
      wait_writes(g - _L, slot_n)
      fire_gathers(g + _L, slot_n)
      wait_gathers(g, b)
      fire_writes(g, b)
    return carry

  lax.fori_loop(1, ng // _S - 1, outer, 0)

  t_last = ng // _S - 1
  for b in range(_S):
    g = t_last * _S + b
    slot_n = (b + _L) % _S
    if g + _L < ng:
      wait_writes(g - _L, slot_n)
      fire_gathers(g + _L, slot_n)
    wait_gathers(g, b)
    fire_writes(g, b)
  for b in range(_S):
    wait_writes(ng - _S + b, b)


def _hop1_body(n_per_w, ent_hbm, adj_e_hbm, adj_r_hbm, e_out, r_out,
               idx_v, ebuf, rbuf, *sems):
  gsems, wsems = sems[:_S], sems[_S:]
  wid = lax.axis_index("s") * _NC + lax.axis_index("c")
  base = wid * n_per_w
  pltpu.sync_copy(ent_hbm.at[pl.ds(base, n_per_w)], idx_v)

  def fire_gathers(g, slot):
    off = idx_v.at[pl.ds(g * _G, _G)]
    pltpu.async_copy(adj_e_hbm.at[off], ebuf.at[slot], gsems[slot])
    pltpu.async_copy(adj_r_hbm.at[off], rbuf.at[slot], gsems[slot])

  def wait_gathers(g, slot):
    out0 = base + g * _G
    pltpu.make_async_copy(
        e_out.at[pl.ds(out0, _G)], ebuf.at[slot], gsems[slot]).wait()
    pltpu.make_async_copy(
        r_out.at[pl.ds(out0, _G)], rbuf.at[slot], gsems[slot]).wait()

  def fire_writes(g, slot):
    out0 = base + g * _G
    pltpu.async_copy(ebuf.at[slot], e_out.at[pl.ds(out0, _G)], wsems[slot])
    pltpu.async_copy(rbuf.at[slot], r_out.at[pl.ds(out0, _G)], wsems[slot])

  def wait_writes(g, slot):
    out0 = base + g * _G
    pltpu.make_async_copy(
        ebuf.at[slot], e_out.at[pl.ds(out0, _G)], wsems[slot]).wait()
    pltpu.make_async_copy(
        rbuf.at[slot], r_out.at[pl.ds(out0, _G)], wsems[slot]).wait()

  _ring_pipeline(n_per_w // _G, fire_gathers, wait_gathers,
                 fire_writes, wait_writes)


def _hop2_body(n_per_w, idx_hbm, adj_hbm, out, idx_v, buf, *sems):
  gsems, wsems = sems[:_S], sems[_S:]
  wid = lax.axis_index("s") * _NC + lax.axis_index("c")
  base = wid * n_per_w
  pltpu.sync_copy(idx_hbm.at[pl.ds(base, n_per_w)], idx_v)

  def fire_gathers(g, slot):
    off = idx_v.at[pl.ds(g * _G, _G)]
    pltpu.async_copy(adj_hbm.at[off], buf.at[slot], gsems[slot])

  def wait_gathers(g, slot):
    pltpu.make_async_copy(
        out.at[pl.ds(base + g * _G, _G)], buf.at[slot], gsems[slot]).wait()

  def fire_writes(g, slot):
    pltpu.async_copy(buf.at[slot], out.at[pl.ds(base + g * _G, _G)],
                     wsems[slot])

  def wait_writes(g, slot):
    pltpu.make_async_copy(
        buf.at[slot], out.at[pl.ds(base + g * _G, _G)], wsems[slot]).wait()

  _ring_pipeline(n_per_w // _G, fire_gathers, wait_gathers,
                 fire_writes, wait_writes)


_PARAMS = pltpu.CompilerParams(use_tc_tiling_on_sc=False)


def _hop1(entity_flat, adj_entity, adj_relation):
  n = entity_flat.shape[0]
  n_per_w = n // _NW
  body = functools.partial(_hop1_body, n_per_w)
  out_type = (
      jax.ShapeDtypeStruct((n, _NB), jnp.int32),
      jax.ShapeDtypeStruct((n, _NB), jnp.int32),
  )
  scratch = [
      pltpu.VMEM((n_per_w,), jnp.int32),
      pltpu.VMEM((_S, _G, _NB), jnp.int32),
      pltpu.VMEM((_S, _G, _NB), jnp.int32),
  ] + [pltpu.SemaphoreType.DMA] * (2 * _S)
  return pl.kernel(
      body, out_type=out_type, mesh=_mesh(), scratch_types=scratch,
      compiler_params=_PARAMS)(entity_flat, adj_entity, adj_relation)


def _hop2(idx_flat, adj):
  n = idx_flat.shape[0]
  n_per_w = n // _NW
  body = functools.partial(_hop2_body, n_per_w)
  out_type = jax.ShapeDtypeStruct((n, _NB), jnp.int32)
  scratch = [
      pltpu.VMEM((n_per_w,), jnp.int32),
      pltpu.VMEM((_S, _G, _NB), jnp.int32),
  ] + [pltpu.SemaphoreType.DMA] * (2 * _S)
  return pl.kernel(
      body, out_type=out_type, mesh=_mesh(), scratch_types=scratch,
      compiler_params=_PARAMS)(idx_flat, adj)


def kernel(entity, adj_entity, adj_relation):
  ent1, rel1 = _hop1(entity.reshape(-1), adj_entity, adj_relation)
  idx2 = ent1.reshape(-1)
  ent2 = _hop2(idx2, adj_entity)
  rel2 = _hop2(idx2, adj_relation)
  return (entity,
          ent1,
          ent2.reshape(_BATCH, _NB * _NB),
          rel1,
          rel2.reshape(_BATCH, _NB * _NB))


# speedup vs baseline: 1.0624x; 1.0037x over previous
import functools

import jax
import jax.numpy as jnp
from jax import lax
from jax.experimental import pallas as pl
from jax.experimental.pallas import tpu as pltpu
from jax.experimental.pallas import tpu_sc as plsc

_NB = 32
_BATCH = 16384
_NC = 2
_NS = 16
_NW = _NC * _NS
_G = 128
_S2 = 8


def _mesh():
  return plsc.VectorSubcoreMesh(
      core_axis_name="c", subcore_axis_name="s",
      num_cores=_NC, num_subcores=_NS)


def _ring_pipeline(ng, S, fire_gathers, wait_gathers, fire_writes,
                   wait_writes):
  L = S // 2
  if ng <= S:
    for g in range(ng):
      fire_gathers(g, g)
    for g in range(ng):
      wait_gathers(g, g)
      fire_writes(g, g)
    for g in range(ng):
      wait_writes(g, g)
    return

  for b in range(L):
    fire_gathers(b, b)
  for b in range(S):
    g = b
    slot_n = (b + L) % S
    if g + L >= S:
      wait_writes(g - L, slot_n)
    fire_gathers(g + L, slot_n)
    wait_gathers(g, b)
    fire_writes(g, b)

  def outer(t, carry):
    for b in range(S):
      g = t * S + b
      slot_n = (b + L) % S
      wait_writes(g - L, slot_n)
      fire_gathers(g + L, slot_n)
      wait_gathers(g, b)
      fire_writes(g, b)
    return carry

  lax.fori_loop(1, ng // S - 1, outer, 0)

  t_last = ng // S - 1
  for b in range(S):
    g = t_last * S + b
    slot_n = (b + L) % S
    if g + L < ng:
      wait_writes(g - L, slot_n)
      fire_gathers(g + L, slot_n)
    wait_gathers(g, b)
    fire_writes(g, b)
  for b in range(S):
    wait_writes(ng - S + b, b)


def _hop2_body(n_per_w, idx_hbm, adj_hbm, out, idx_v, buf, *sems):
  gsems, wsems = sems[:_S2], sems[_S2:]
  wid = lax.axis_index("s") * _NC + lax.axis_index("c")
  base = wid * n_per_w
  pltpu.sync_copy(idx_hbm.at[pl.ds(base, n_per_w)], idx_v)

  def fire_gathers(g, slot):
    off = idx_v.at[pl.ds(g * _G, _G)]
    pltpu.async_copy(adj_hbm.at[off], buf.at[slot], gsems[slot])

  def wait_gathers(g, slot):
    off = idx_v.at[pl.ds(g * _G, _G)]
    pltpu.make_async_copy(adj_hbm.at[off], buf.at[slot], gsems[slot]).wait()

  def out_slice(g):
    return out.at[pl.ds(base + g * _G, _G)]

  def fire_writes(g, slot):
    pltpu.async_copy(buf.at[slot], out_slice(g), wsems[slot])

  def wait_writes(g, slot):
    pltpu.make_async_copy(buf.at[slot], out_slice(g), wsems[slot]).wait()

  _ring_pipeline(n_per_w // _G, _S2, fire_gathers, wait_gathers,
                 fire_writes, wait_writes)


_PARAMS = pltpu.CompilerParams(use_tc_tiling_on_sc=False)


def _hop2(idx_flat, adj):
  n = idx_flat.shape[0]
  n_per_w = n // _NW
  body = functools.partial(_hop2_body, n_per_w)
  out_type = jax.ShapeDtypeStruct((n, _NB), jnp.int32)
  scratch = [
      pltpu.VMEM((n_per_w,), jnp.int32),
      pltpu.VMEM((_S2, _G, _NB), jnp.int32),
  ] + [pltpu.SemaphoreType.DMA] * (2 * _S2)
  return pl.kernel(
      body, out_type=out_type, mesh=_mesh(), scratch_types=scratch,
      compiler_params=_PARAMS)(idx_flat, adj)


def kernel(entity, adj_entity, adj_relation):
  ent_flat = entity.reshape(-1)
  ent1 = _hop2(ent_flat, adj_entity)
  idx2 = ent1.reshape(-1)
  ent2 = _hop2(idx2, adj_entity)
  rel1 = _hop2(ent_flat, adj_relation)
  rel2 = _hop2(idx2, adj_relation)
  return (entity,
          ent1,
          ent2.reshape(_BATCH, _NB * _NB),
          rel1,
          rel2.reshape(_BATCH, _NB * _NB))
